# baseline (speedup 1.0000x reference)
import functools

import jax
import jax.numpy as jnp
from jax import lax
from jax.experimental import pallas as pl
from jax.experimental.pallas import tpu as pltpu
from jax.experimental.pallas import tpu_sc as plsc

N = 10000
E = 160000
DF = 128
DE = 16
NR = 4
H = 4
HD = 64
D = H * HD
IDED = 32
NG = 16
NC = 10
KH = 128
GS = 5
SO = 3
FH = 128

NCORE = 2
NSUB = 16
NW = NCORE * NSUB
EP = 163840
NP = 10240
DK = NR * N
CH = 128

@functools.lru_cache(maxsize=None)
def _mesh():
    return plsc.VectorSubcoreMesh(
        core_axis_name="c", subcore_axis_name="s",
        num_cores=NCORE, num_subcores=NSUB)


def _silu(v):
    return v * jax.nn.sigmoid(v)


def _ln(v, g, b):
    m = jnp.mean(v, axis=-1, keepdims=True)
    var = jnp.mean((v - m) ** 2, axis=-1, keepdims=True)
    return (v - m) / jnp.sqrt(var + 1e-5) * g + b


@functools.lru_cache(maxsize=None)
def _build_sc_gather_ide():
  @functools.partial(
    pl.kernel,
    out_type=jax.ShapeDtypeStruct((NP, IDED), jnp.float32),
    mesh=_mesh(),
    compiler_params=pltpu.CompilerParams(use_tc_tiling_on_sc=False),
    scratch_types=[pltpu.VMEM((64,), jnp.int32),
                   pltpu.VMEM((64, IDED), jnp.float32),
                   pltpu.SemaphoreType.DMA],
  )
  def _k(emb_hbm, idx_hbm, out_hbm, idx_v, rows_v, sem):
    wid = lax.axis_index("s") * NCORE + lax.axis_index("c")
    wbase = wid * (NP // NW)

    def body(i, carry):
        base = wbase + i * 64
        pltpu.sync_copy(idx_hbm.at[pl.ds(base, 64)], idx_v)
        pltpu.async_copy(emb_hbm.at[idx_v], rows_v, sem).wait()
        pltpu.sync_copy(rows_v, out_hbm.at[pl.ds(base, 64)])
        return carry

    lax.fori_loop(0, NP // NW // 64, body, 0)
  return _k


def _sc_gather_ide(emb, idtok_pad):
    return _build_sc_gather_ide()(emb, idtok_pad)


WPE = EP // NW


@functools.lru_cache(maxsize=None)
def _build_sc_gather_xlxr():
  @functools.partial(
    pl.kernel,
    out_type=(jax.ShapeDtypeStruct((EP, D // 2), jnp.int32),
              jax.ShapeDtypeStruct((EP, D // 2), jnp.int32),
              jax.ShapeDtypeStruct((EP,), jnp.int32)),
    mesh=_mesh(),
    compiler_params=pltpu.CompilerParams(use_tc_tiling_on_sc=True),
    scratch_types=[pltpu.VMEM((WPE,), jnp.int32),
                   pltpu.VMEM((WPE,), jnp.int32),
                   pltpu.VMEM((WPE,), jnp.int32),
                   pltpu.VMEM((CH, D // 2), jnp.int32),
                   pltpu.VMEM((CH, D // 2), jnp.int32),
                   pltpu.VMEM((CH, D // 2), jnp.int32),
                   pltpu.VMEM((CH, D // 2), jnp.int32),
                   pltpu.SemaphoreType.DMA,
                   pltpu.SemaphoreType.DMA,
                   pltpu.SemaphoreType.DMA,
                   pltpu.SemaphoreType.DMA,
                   pltpu.SemaphoreType.DMA,
                   pltpu.SemaphoreType.DMA,
                   pltpu.SemaphoreType.DMA,
                   pltpu.SemaphoreType.DMA],
  )
  def _k(xl4, xr4, src_hbm, dst_hbm, et_hbm, xlg, xrg, dkey_out,
         skey_a, dkey_a, et_a, xlb0, xlb1, xrb0, xrb1,
         g0, g1, h0, h1, wl0, wl1, wr0, wr1):
    wid = lax.axis_index("s") * NCORE + lax.axis_index("c")
    wbase = wid * WPE
    xlb = [xlb0, xlb1]
    xrb = [xrb0, xrb1]
    gsem = [g0, g1]
    hsem = [h0, h1]
    wlsem = [wl0, wl1]
    wrsem = [wr0, wr1]

    pltpu.sync_copy(src_hbm.at[pl.ds(wbase, WPE)], skey_a)
    pltpu.sync_copy(dst_hbm.at[pl.ds(wbase, WPE)], dkey_a)
    pltpu.sync_copy(et_hbm.at[pl.ds(wbase, WPE)], et_a)

    def keys(j, kc):
        sl = pl.ds(j * 16, 16)
        etn = et_a[sl] * N
        skey_a[sl] = skey_a[sl] + etn
        dkey_a[sl] = dkey_a[sl] + etn
        return kc

    lax.fori_loop(0, WPE // 16, keys, 0)
    pltpu.sync_copy(dkey_a, dkey_out.at[pl.ds(wbase, WPE)])

    def pair(p, carry):
        dxl, dxr = [], []
        for b in range(2):
            i = p * 2 + b

            @pl.when(p > 0)
            def _drain():
                pltpu.make_async_copy(
                    xlb[b], xlg.at[pl.ds(wbase, CH)], wlsem[b]).wait()
                pltpu.make_async_copy(
                    xrb[b], xrg.at[pl.ds(wbase, CH)], wrsem[b]).wait()

            dxl.append(pltpu.async_copy(
                xl4.at[skey_a.at[pl.ds(i * CH, CH)]], xlb[b], gsem[b]))
            dxr.append(pltpu.async_copy(
                xr4.at[dkey_a.at[pl.ds(i * CH, CH)]], xrb[b], hsem[b]))
        for b in range(2):
            i = p * 2 + b
            base = wbase + i * CH
            dxl[b].wait()
            pltpu.async_copy(xlb[b], xlg.at[pl.ds(base, CH)], wlsem[b])
            dxr[b].wait()
            pltpu.async_copy(xrb[b], xrg.at[pl.ds(base, CH)], wrsem[b])
        return carry

    lax.fori_loop(0, WPE // CH // 2, pair, 0)
    for b in range(2):
        pltpu.make_async_copy(
            xlb[b], xlg.at[pl.ds(wbase, CH)], wlsem[b]).wait()
        pltpu.make_async_copy(
            xrb[b], xrg.at[pl.ds(wbase, CH)], wrsem[b]).wait()
  return _k


def _sc_gather_xlxr(xl4, xr4, src_pad, dst_pad, et_pad):
    return _build_sc_gather_xlxr()(xl4, xr4, src_pad, dst_pad, et_pad)


@functools.lru_cache(maxsize=None)
def _build_sc_scatter_den():
  @functools.partial(
    pl.kernel,
    out_type=jax.ShapeDtypeStruct((NCORE, DK, 16), jnp.float32),
    mesh=_mesh(),
    compiler_params=pltpu.CompilerParams(use_tc_tiling_on_sc=False),
    scratch_types=[pltpu.VMEM((CH, 16), jnp.float32),
                   pltpu.VMEM((CH,), jnp.int32),
                   pltpu.VMEM_SHARED((DK, 16), jnp.float32)],
  )
  def _k(pe_hbm, dkey_hbm, zeros_hbm, out_hbm, pe_v, key_v, den_sh):
    cid = lax.axis_index("c")
    sid = lax.axis_index("s")
    wid = sid * NCORE + cid
    rows = DK // NSUB
    pltpu.sync_copy(zeros_hbm.at[pl.ds(sid * rows, rows)],
                    den_sh.at[pl.ds(sid * rows, rows)])
    plsc.subcore_barrier()
    wbase = wid * (EP // NW)

    def body(i, carry):
        base = wbase + i * CH
        pltpu.sync_copy(pe_hbm.at[pl.ds(base, CH)], pe_v)
        pltpu.sync_copy(dkey_hbm.at[pl.ds(base, CH)], key_v)
        pltpu.sync_copy(pe_v, den_sh.at[key_v], add=True)
        return carry

    lax.fori_loop(0, EP // NW // CH, body, 0)
    plsc.subcore_barrier()
    pltpu.sync_copy(den_sh.at[pl.ds(sid * rows, rows)],
                    out_hbm.at[cid, pl.ds(sid * rows, rows)])
  return _k


def _sc_scatter_den(pe, dkey, den_zeros):
    return _build_sc_scatter_den()(pe, dkey, den_zeros)


@functools.lru_cache(maxsize=None)
def _build_sc_gather_den():
  @functools.partial(
    pl.kernel,
    out_type=(jax.ShapeDtypeStruct((EP, 16), jnp.float32),
              jax.ShapeDtypeStruct((EP, 16), jnp.float32)),
    mesh=_mesh(),
    compiler_params=pltpu.CompilerParams(use_tc_tiling_on_sc=False),
    scratch_types=[pltpu.VMEM((CH,), jnp.int32),
                   pltpu.VMEM((CH, 16), jnp.float32),
                   pltpu.VMEM((CH, 16), jnp.float32),
                   pltpu.SemaphoreType.DMA,
                   pltpu.SemaphoreType.DMA],
  )
  def _k(denp0, denp1, dkey_hbm, out0, out1,
         key_v, d0_v, d1_v, sem1, sem2):
    wid = lax.axis_index("s") * NCORE + lax.axis_index("c")
    wbase = wid * (EP // NW)

    def body(i, carry):
        base = wbase + i * CH
        pltpu.sync_copy(dkey_hbm.at[pl.ds(base, CH)], key_v)
        g1 = pltpu.async_copy(denp0.at[key_v], d0_v, sem1)
        g2 = pltpu.async_copy(denp1.at[key_v], d1_v, sem2)
        g1.wait()
        pltpu.sync_copy(d0_v, out0.at[pl.ds(base, CH)])
        g2.wait()
        pltpu.sync_copy(d1_v, out1.at[pl.ds(base, CH)])
        return carry

    lax.fori_loop(0, EP // NW // CH, body, 0)
  return _k


def _sc_gather_den(denp0, denp1, dkey):
    return _build_sc_gather_den()(denp0, denp1, dkey)


@functools.lru_cache(maxsize=None)
def _build_sc_scatter_hm():
  @functools.partial(
    pl.kernel,
    out_type=jax.ShapeDtypeStruct((NCORE, N, HD), jnp.float32),
    mesh=_mesh(),
    compiler_params=pltpu.CompilerParams(use_tc_tiling_on_sc=False),
    scratch_types=[pltpu.VMEM((CH, HD), jnp.float32),
                   pltpu.VMEM((CH,), jnp.int32),
                   pltpu.VMEM_SHARED((N, HD), jnp.float32)],
  )
  def _k(v_hbm, dst_hbm, zeros_hbm, out_hbm, v_v, key_v, hm_sh):
    cid = lax.axis_index("c")
    sid = lax.axis_index("s")
    wid = sid * NCORE + cid
    rows = N // NSUB
    pltpu.sync_copy(zeros_hbm.at[pl.ds(sid * rows, rows)],
                    hm_sh.at[pl.ds(sid * rows, rows)])
    plsc.subcore_barrier()
    wbase = wid * (EP // NW)

    def body(i, carry):
        base = wbase + i * CH
        pltpu.sync_copy(v_hbm.at[pl.ds(base, CH)], v_v)
        pltpu.sync_copy(dst_hbm.at[pl.ds(base, CH)], key_v)
        pltpu.sync_copy(v_v, hm_sh.at[key_v], add=True)
        return carry

    lax.fori_loop(0, EP // NW // CH, body, 0)
    plsc.subcore_barrier()
    pltpu.sync_copy(hm_sh.at[pl.ds(sid * rows, rows)],
                    out_hbm.at[cid, pl.ds(sid * rows, rows)])
  return _k


def _sc_scatter_hm(v, dst_pad, hm_zeros):
    return _build_sc_scatter_hm()(v, dst_pad, hm_zeros)


BT = 1000
BE = 1024


def _h0_body(x_ref, ide_ref, wpt_ref, bp_ref, o_ref):
    wpt = wpt_ref[...]
    z = (jnp.dot(x_ref[...], wpt[:DF], preferred_element_type=jnp.float32)
         + jnp.dot(ide_ref[...], wpt[DF:], preferred_element_type=jnp.float32)
         + bp_ref[...])
    o_ref[...] = _silu(z)


def _tc_h0(x, ide, wpt, bp2d):
    return pl.pallas_call(
        _h0_body,
        grid=(N // BT,),
        in_specs=[pl.BlockSpec((BT, DF), lambda i: (i, 0)),
                  pl.BlockSpec((BT, IDED), lambda i: (i, 0)),
                  pl.BlockSpec((DF + IDED, HD), lambda i: (0, 0)),
                  pl.BlockSpec((1, HD), lambda i: (0, 0))],
        out_specs=pl.BlockSpec((BT, HD), lambda i: (i, 0)),
        out_shape=jax.ShapeDtypeStruct((N, HD), jnp.float32),
    )(x, ide, wpt, bp2d)


def _proj_body(h_ref, wl_ref, wr_ref, xl_ref, xr_ref):
    h = h_ref[...]
    xl_ref[0] = jnp.dot(h, wl_ref[0],
                        preferred_element_type=jnp.float32).astype(jnp.bfloat16)
    xr_ref[0] = jnp.dot(h, wr_ref[0],
                        preferred_element_type=jnp.float32).astype(jnp.bfloat16)


def _tc_proj(h0, wlt, wrt):
    BTP = 2000
    return pl.pallas_call(
        _proj_body,
        grid=(NR, N // BTP),
        in_specs=[pl.BlockSpec((BTP, HD), lambda r, i: (i, 0)),
                  pl.BlockSpec((1, HD, D), lambda r, i: (r, 0, 0)),
                  pl.BlockSpec((1, HD, D), lambda r, i: (r, 0, 0))],
        out_specs=[pl.BlockSpec((1, BTP, D), lambda r, i: (r, i, 0)),
                   pl.BlockSpec((1, BTP, D), lambda r, i: (r, i, 0))],
        out_shape=(jax.ShapeDtypeStruct((NR, N, D), jnp.bfloat16),
                   jax.ShapeDtypeStruct((NR, N, D), jnp.bfloat16)),
    )(h0, wlt, wrt)


def _ef_body(ea_ref, wa_ref, rel_ref, wr_ref, et_ref, o_ref):
    dn = (((0,), (0,)), ((), ()))
    ef_all = lax.dot_general(ea_ref[...], wa_ref[...], dn,
                             preferred_element_type=jnp.float32)
    r4 = jnp.dot(rel_ref[...], wr_ref[...],
                 preferred_element_type=jnp.float32)
    et = et_ref[...]
    acc = jnp.zeros((ea_ref.shape[1], D), jnp.float32)
    for r in range(NR):
        sel = ef_all[:, r * D:(r + 1) * D] + r4[r, r * D:(r + 1) * D][None]
        acc = acc + jnp.where(et == r, sel, 0.0)
    o_ref[...] = acc.astype(jnp.bfloat16)


def _tc_ef(eat_pad, wa, rel_emb, wrel, et2d):
    return pl.pallas_call(
        _ef_body,
        grid=(EP // BE,),
        in_specs=[pl.BlockSpec((DE, BE), lambda i: (0, i)),
                  pl.BlockSpec((DE, NR * D), lambda i: (0, 0)),
                  pl.BlockSpec((NR, 8), lambda i: (0, 0)),
                  pl.BlockSpec((8, NR * D), lambda i: (0, 0)),
                  pl.BlockSpec((BE, 1), lambda i: (i, 0))],
        out_specs=pl.BlockSpec((BE, D), lambda i: (i, 0)),
        out_shape=jax.ShapeDtypeStruct((EP, D), jnp.bfloat16),
    )(eat_pad, wa, rel_emb, wrel, et2d)


def _scores_body(xlg_ref, xrg_ref, ef_ref, et_ref, att_ref, o_ref):
    pid = pl.program_id(0)
    z = (xlg_ref[...].astype(jnp.float32) + xrg_ref[...].astype(jnp.float32)
         + ef_ref[...].astype(jnp.float32))
    z = jnp.where(z >= 0, z, 0.2 * z)
    et = et_ref[...]
    rr = lax.broadcasted_iota(jnp.int32, (BE, NR), 1)
    oh = (et == rr).astype(jnp.float32)
    asel = jnp.dot(oh, att_ref[...], preferred_element_type=jnp.float32)
    prod = z * asel
    f64 = lax.broadcasted_iota(jnp.int32, (D, H), 0) // HD
    hh = lax.broadcasted_iota(jnp.int32, (D, H), 1)
    hmask = (f64 == hh).astype(jnp.float32)
    e = jnp.dot(prod, hmask, preferred_element_type=jnp.float32)
    pe = jnp.exp(e)
    rowid = pid * BE + lax.broadcasted_iota(jnp.int32, (BE, 1), 0)
    pe = jnp.where(rowid < E, pe, 0.0)
    o_ref[...] = jnp.concatenate(
        [pe, jnp.zeros((BE, 16 - H), jnp.float32)], axis=1)


def _tc_scores(xlg, xrg, ef, et2d, att4):
    return pl.pallas_call(
        _scores_body,
        grid=(EP // BE,),
        in_specs=[pl.BlockSpec((BE, D), lambda i: (i, 0)),
                  pl.BlockSpec((BE, D), lambda i: (i, 0)),
                  pl.BlockSpec((BE, D), lambda i: (i, 0)),
                  pl.BlockSpec((BE, 1), lambda i: (i, 0)),
                  pl.BlockSpec((NR, D), lambda i: (0, 0))],
        out_specs=pl.BlockSpec((BE, 16), lambda i: (i, 0)),
        out_shape=jax.ShapeDtypeStruct((EP, 16), jnp.float32),
    )(xlg, xrg, ef, et2d, att4)


def _values_body(xlg_ref, pe_ref, dg0_ref, dg1_ref, et_ref, rg_ref, o_ref):
    pe = pe_ref[...][:, :H]
    den = dg0_ref[...][:, :H] + dg1_ref[...][:, :H]
    den = jnp.where(den > 0, den, 1.0)
    alpha = pe / den
    rg = rg_ref[...]
    gexp = jnp.exp(rg - jnp.max(rg, axis=1, keepdims=True))
    gw = gexp / jnp.sum(gexp, axis=1, keepdims=True)
    et = et_ref[...]
    rr = lax.broadcasted_iota(jnp.int32, (BE, NR), 1)
    oh = (et == rr).astype(jnp.float32)
    gws = jnp.dot(oh, jnp.reshape(gw, (NR, 1)),
                  preferred_element_type=jnp.float32)
    w = alpha * gws * (1.0 / H)
    xlg = xlg_ref[...].astype(jnp.float32)
    v = jnp.zeros((BE, HD), jnp.float32)
    for h in range(H):
        v = v + w[:, h:h + 1] * xlg[:, h * HD:(h + 1) * HD]
    o_ref[...] = v


def _tc_values(xlg, pe, dg0, dg1, et2d, rg2d):
    return pl.pallas_call(
        _values_body,
        grid=(EP // BE,),
        in_specs=[pl.BlockSpec((BE, D), lambda i: (i, 0)),
                  pl.BlockSpec((BE, 16), lambda i: (i, 0)),
                  pl.BlockSpec((BE, 16), lambda i: (i, 0)),
                  pl.BlockSpec((BE, 16), lambda i: (i, 0)),
                  pl.BlockSpec((BE, 1), lambda i: (i, 0)),
                  pl.BlockSpec((1, NR), lambda i: (0, 0))],
        out_specs=pl.BlockSpec((BE, HD), lambda i: (i, 0)),
        out_shape=jax.ShapeDtypeStruct((EP, HD), jnp.float32),
    )(xlg, pe, dg0, dg1, et2d, rg2d)


def _post_body(h0_ref, hmp_ref, batch_ref, rg_ref, bias4_ref,
               n1g_ref, n1b_ref, w1t_ref, b1_ref, w2t_ref, b2_ref,
               n2g_ref, n2b_ref, sums_ref, cnt_ref, maxr_ref):
    pid = pl.program_id(0)

    @pl.when(pid == 0)
    def _init():
        sums_ref[...] = jnp.zeros_like(sums_ref)
        cnt_ref[...] = jnp.zeros_like(cnt_ref)
        maxr_ref[...] = jnp.full_like(maxr_ref, -jnp.inf)

    rg = rg_ref[...]
    gexp = jnp.exp(rg - jnp.max(rg, axis=1, keepdims=True))
    gw = gexp / jnp.sum(gexp, axis=1, keepdims=True)
    hmc = jnp.dot(gw, bias4_ref[...], preferred_element_type=jnp.float32)
    h = h0_ref[...] + hmp_ref[0] + hmp_ref[1] + hmc
    h = _ln(h, n1g_ref[...], n1b_ref[...])
    hf = jnp.dot(_silu(jnp.dot(h, w1t_ref[...],
                               preferred_element_type=jnp.float32)
                       + b1_ref[...]),
                 w2t_ref[...], preferred_element_type=jnp.float32) + b2_ref[...]
    h2 = _ln(h + hf, n2g_ref[...], n2b_ref[...])

    batch = batch_ref[...]
    gg = lax.broadcasted_iota(jnp.int32, (BT, NG), 1)
    oh = (batch == gg).astype(jnp.float32)
    dn = (((0,), (0,)), ((), ()))
    sums_ref[...] += lax.dot_general(oh, h2, dn,
                                     preferred_element_type=jnp.float32)
    cnt_ref[...] += lax.dot_general(oh, jnp.ones((BT, HD), jnp.float32), dn,
                                    preferred_element_type=jnp.float32)
    for g in range(NG):
        mg = jnp.where(batch == g, h2, -jnp.inf)
        maxr_ref[g:g + 1, :] = jnp.maximum(
            maxr_ref[g:g + 1, :], jnp.max(mg, axis=0, keepdims=True))


def _tc_post(h0, hmp, batch2d, rg2d, bias4, n1g, n1b, w1t, b1, w2t, b2,
             n2g, n2b):
    return pl.pallas_call(
        _post_body,
        grid=(N // BT,),
        in_specs=[pl.BlockSpec((BT, HD), lambda i: (i, 0)),
                  pl.BlockSpec((NCORE, BT, HD), lambda i: (0, i, 0)),
                  pl.BlockSpec((BT, 1), lambda i: (i, 0)),
                  pl.BlockSpec((1, NR), lambda i: (0, 0)),
                  pl.BlockSpec((NR, HD), lambda i: (0, 0)),
                  pl.BlockSpec((1, HD), lambda i: (0, 0)),
                  pl.BlockSpec((1, HD), lambda i: (0, 0)),
                  pl.BlockSpec((HD, FH), lambda i: (0, 0)),
                  pl.BlockSpec((1, FH), lambda i: (0, 0)),
                  pl.BlockSpec((FH, HD), lambda i: (0, 0)),
                  pl.BlockSpec((1, HD), lambda i: (0, 0)),
                  pl.BlockSpec((1, HD), lambda i: (0, 0)),
                  pl.BlockSpec((1, HD), lambda i: (0, 0))],
        out_specs=[pl.BlockSpec((NG, HD), lambda i: (0, 0)),
                   pl.BlockSpec((NG, HD), lambda i: (0, 0)),
                   pl.BlockSpec((NG, HD), lambda i: (0, 0))],
        out_shape=(jax.ShapeDtypeStruct((NG, HD), jnp.float32),
                   jax.ShapeDtypeStruct((NG, HD), jnp.float32),
                   jax.ShapeDtypeStruct((NG, HD), jnp.float32)),
    )(h0, hmp, batch2d, rg2d, bias4, n1g, n1b, w1t, b1, w2t, b2, n2g, n2b)


def _kan_eval(gmat, bwt, gridt, sct):
    base = jnp.dot(_silu(gmat), bwt, preferred_element_type=jnp.float32)
    nb = gridt.shape[0] - 1
    bs = [((gmat >= gridt[j:j + 1]) & (gmat < gridt[j + 1:j + 2]))
          .astype(jnp.float32) for j in range(nb)]
    for deg in range(1, SO + 1):
        nbs = []
        for k in range(nb - deg):
            t1 = (gmat - gridt[k:k + 1]) / (gridt[k + deg:k + deg + 1]
                                            - gridt[k:k + 1])
            t2 = ((gridt[k + deg + 1:k + deg + 2] - gmat)
                  / (gridt[k + deg + 1:k + deg + 2] - gridt[k + 1:k + 2]))
            nbs.append(t1 * bs[k] + t2 * bs[k + 1])
        bs = nbs
    out = base
    for j in range(GS + SO):
        out = out + jnp.dot(bs[j], sct[j], preferred_element_type=jnp.float32)
    return out


def _kan_body(sums_ref, cnt_ref, maxr_ref, rng_ref, rnb_ref,
              bwt1_ref, gridt1_ref, sct1_ref, bwt2_ref, gridt2_ref, sct2_ref,
              o_ref):
    cnt = jnp.maximum(cnt_ref[...], 1.0)
    hmean = sums_ref[...] / cnt
    maxr = maxr_ref[...]
    hmax = jnp.where(maxr < -1e30, 0.0, maxr)
    g0 = jnp.concatenate([hmean, hmax], axis=1)
    g0 = _ln(g0, rng_ref[...], rnb_ref[...])
    g1 = _kan_eval(g0, bwt1_ref[...], gridt1_ref[...], sct1_ref[...])
    g2 = _kan_eval(g1, bwt2_ref[...], gridt2_ref[...], sct2_ref[...])
    o_ref[...] = g2


def _tc_kan(sums, cnt, maxr, rng2d, rnb2d, bwt1, gridt1, sct1,
            bwt2, gridt2, sct2):
    return pl.pallas_call(
        _kan_body,
        out_shape=jax.ShapeDtypeStruct((NG, NC), jnp.float32),
    )(sums, cnt, maxr, rng2d, rnb2d, bwt1, gridt1, sct1, bwt2, gridt2, sct2)


def kernel(x, edge_index, edge_attr, edge_type, id_token, batch, params):
    p = params
    bp0 = p['blocks'][0]
    f32 = jnp.float32

    idtok_pad = jnp.concatenate(
        [id_token.astype(jnp.int32), jnp.zeros((NP - N,), jnp.int32)])
    zpad = jnp.zeros((EP - E,), jnp.int32)
    src_pad = jnp.concatenate([edge_index[0].astype(jnp.int32), zpad])
    dst_pad = jnp.concatenate([edge_index[1].astype(jnp.int32), zpad])
    et_pad = jnp.concatenate([edge_type.astype(jnp.int32), zpad])
    et2d = et_pad.reshape(EP, 1)
    eat_pad = jnp.concatenate(
        [edge_attr.T, jnp.zeros((DE, EP - E), f32)], axis=1)

    wpt = p['Wp'].T
    bp2d = p['bp'].reshape(1, HD)
    wlt = jnp.stack([bp0['convs'][r]['lin_l'].T for r in range(NR)])
    wrt = jnp.stack([bp0['convs'][r]['lin_r'].T for r in range(NR)])
    wa = jnp.concatenate(
        [bp0['convs'][r]['lin_edge'][:, :DE].T for r in range(NR)], axis=1)
    wrel = jnp.concatenate(
        [bp0['convs'][r]['lin_edge'][:, DE:].T for r in range(NR)], axis=1)
    att4 = jnp.stack([bp0['convs'][r]['att'].reshape(D) for r in range(NR)])
    bias4 = jnp.stack([bp0['convs'][r]['bias'] for r in range(NR)])
    rg2d = bp0['rel_gate'].reshape(1, NR)

    den_zeros = jnp.zeros((DK, 16), f32)
    hm_zeros = jnp.zeros((N, HD), f32)

    kan1, kan2 = p['kan'][0], p['kan'][1]
    bwt1 = kan1['bw'].T
    gridt1 = kan1['grid'].T
    sct1 = jnp.transpose(kan1['sw'] * kan1['ss'][..., None], (2, 1, 0))
    bwt2 = kan2['bw'].T
    gridt2 = kan2['grid'].T
    sct2 = jnp.transpose(kan2['sw'] * kan2['ss'][..., None], (2, 1, 0))

    ide = _sc_gather_ide(p['id_emb'], idtok_pad)[:N]
    h0 = _tc_h0(x, ide, wpt, bp2d)
    xl, xr = _tc_proj(h0, wlt, wrt)
    xl4 = lax.bitcast_convert_type(
        xl.reshape(NR * N, D // 2, 2), jnp.int32)
    xr4 = lax.bitcast_convert_type(
        xr.reshape(NR * N, D // 2, 2), jnp.int32)
    ef = _tc_ef(eat_pad, wa, bp0['rel_emb'], wrel, et2d)
    xlgp, xrgp, dkey = _sc_gather_xlxr(xl4, xr4, src_pad, dst_pad, et_pad)
    xlg = lax.bitcast_convert_type(xlgp, jnp.bfloat16).reshape(EP, D)
    xrg = lax.bitcast_convert_type(xrgp, jnp.bfloat16).reshape(EP, D)
    pe = _tc_scores(xlg, xrg, ef, et2d, att4)
    denp = _sc_scatter_den(pe, dkey, den_zeros)
    dg0, dg1 = _sc_gather_den(denp[0], denp[1], dkey)
    v = _tc_values(xlg, pe, dg0, dg1, et2d, rg2d)
    hmp = _sc_scatter_hm(v, dst_pad, hm_zeros)
    sums, cnt, maxr = _tc_post(
        h0, hmp, batch.astype(jnp.int32).reshape(N, 1), rg2d, bias4,
        bp0['n1g'].reshape(1, HD), bp0['n1b'].reshape(1, HD),
        bp0['W1'].T, bp0['b1'].reshape(1, FH),
        bp0['W2'].T, bp0['b2'].reshape(1, HD),
        bp0['n2g'].reshape(1, HD), bp0['n2b'].reshape(1, HD))
    out = _tc_kan(sums, cnt, maxr,
                  p['rng'].reshape(1, 2 * HD), p['rnb'].reshape(1, 2 * HD),
                  bwt1, gridt1, sct1, bwt2, gridt2, sct2)
    return out


# speedup vs baseline: 1.9187x; 1.9187x over previous
import functools

import jax
import jax.numpy as jnp
from jax import lax
from jax.experimental import pallas as pl
from jax.experimental.pallas import tpu as pltpu
from jax.experimental.pallas import tpu_sc as plsc

N = 10000
E = 160000
DF = 128
DE = 16
NR = 4
H = 4
HD = 64
D = H * HD
IDED = 32
NG = 16
NC = 10
KH = 128
GS = 5
SO = 3
FH = 128

NCORE = 2
NSUB = 16
NW = NCORE * NSUB
EP = 163840
NP = 10240
DK = NR * N
CH = 128

@functools.lru_cache(maxsize=None)
def _mesh():
    return plsc.VectorSubcoreMesh(
        core_axis_name="c", subcore_axis_name="s",
        num_cores=NCORE, num_subcores=NSUB)


def _silu(v):
    return v * jax.nn.sigmoid(v)


def _ln(v, g, b):
    m = jnp.mean(v, axis=-1, keepdims=True)
    var = jnp.mean((v - m) ** 2, axis=-1, keepdims=True)
    return (v - m) / jnp.sqrt(var + 1e-5) * g + b


@functools.lru_cache(maxsize=None)
def _build_sc_gather_ide():
  @functools.partial(
    pl.kernel,
    out_type=jax.ShapeDtypeStruct((NP, IDED), jnp.float32),
    mesh=_mesh(),
    compiler_params=pltpu.CompilerParams(use_tc_tiling_on_sc=False),
    scratch_types=[pltpu.VMEM((64,), jnp.int32),
                   pltpu.VMEM((64, IDED), jnp.float32),
                   pltpu.SemaphoreType.DMA],
  )
  def _k(emb_hbm, idx_hbm, out_hbm, idx_v, rows_v, sem):
    wid = lax.axis_index("s") * NCORE + lax.axis_index("c")
    wbase = wid * (NP // NW)

    def body(i, carry):
        base = wbase + i * 64
        pltpu.sync_copy(idx_hbm.at[pl.ds(base, 64)], idx_v)
        pltpu.async_copy(emb_hbm.at[idx_v], rows_v, sem).wait()
        pltpu.sync_copy(rows_v, out_hbm.at[pl.ds(base, 64)])
        return carry

    lax.fori_loop(0, NP // NW // 64, body, 0)
  return _k


def _sc_gather_ide(emb, idtok_pad):
    return _build_sc_gather_ide()(emb, idtok_pad)


WPE = EP // NW


@functools.lru_cache(maxsize=None)
def _build_sc_gather_xlxr():
  @functools.partial(
    pl.kernel,
    out_type=(jax.ShapeDtypeStruct((EP, D), jnp.bfloat16),
              jax.ShapeDtypeStruct((EP, D), jnp.bfloat16),
              jax.ShapeDtypeStruct((EP,), jnp.int32)),
    mesh=_mesh(),
    compiler_params=pltpu.CompilerParams(use_tc_tiling_on_sc=False),
    scratch_types=[pltpu.VMEM((WPE,), jnp.int32),
                   pltpu.VMEM((WPE,), jnp.int32),
                   pltpu.VMEM((WPE,), jnp.int32),
                   pltpu.VMEM((CH, D), jnp.bfloat16),
                   pltpu.VMEM((CH, D), jnp.bfloat16),
                   pltpu.VMEM((CH, D), jnp.bfloat16),
                   pltpu.VMEM((CH, D), jnp.bfloat16),
                   pltpu.SemaphoreType.DMA,
                   pltpu.SemaphoreType.DMA,
                   pltpu.SemaphoreType.DMA,
                   pltpu.SemaphoreType.DMA,
                   pltpu.SemaphoreType.DMA,
                   pltpu.SemaphoreType.DMA,
                   pltpu.SemaphoreType.DMA,
                   pltpu.SemaphoreType.DMA],
  )
  def _k(xl4, xr4, src_hbm, dst_hbm, et_hbm, xlg, xrg, dkey_out,
         skey_a, dkey_a, et_a, xlb0, xlb1, xrb0, xrb1,
         g0, g1, h0, h1, wl0, wl1, wr0, wr1):
    wid = lax.axis_index("s") * NCORE + lax.axis_index("c")
    wbase = wid * WPE
    xlb = [xlb0, xlb1]
    xrb = [xrb0, xrb1]
    gsem = [g0, g1]
    hsem = [h0, h1]
    wlsem = [wl0, wl1]
    wrsem = [wr0, wr1]

    pltpu.sync_copy(src_hbm.at[pl.ds(wbase, WPE)], skey_a)
    pltpu.sync_copy(dst_hbm.at[pl.ds(wbase, WPE)], dkey_a)
    pltpu.sync_copy(et_hbm.at[pl.ds(wbase, WPE)], et_a)

    def keys(j, kc):
        sl = pl.ds(j * 16, 16)
        etn = et_a[sl] * N
        skey_a[sl] = skey_a[sl] + etn
        dkey_a[sl] = dkey_a[sl] + etn
        return kc

    lax.fori_loop(0, WPE // 16, keys, 0)
    pltpu.sync_copy(dkey_a, dkey_out.at[pl.ds(wbase, WPE)])

    def pair(p, carry):
        dxl, dxr = [], []
        for b in range(2):
            i = p * 2 + b

            @pl.when(p > 0)
            def _drain():
                pltpu.make_async_copy(
                    xlb[b], xlg.at[pl.ds(wbase, CH)], wlsem[b]).wait()
                pltpu.make_async_copy(
                    xrb[b], xrg.at[pl.ds(wbase, CH)], wrsem[b]).wait()

            dxl.append(pltpu.async_copy(
                xl4.at[skey_a.at[pl.ds(i * CH, CH)]], xlb[b], gsem[b]))
            dxr.append(pltpu.async_copy(
                xr4.at[dkey_a.at[pl.ds(i * CH, CH)]], xrb[b], hsem[b]))
        for b in range(2):
            i = p * 2 + b
            base = wbase + i * CH
            dxl[b].wait()
            pltpu.async_copy(xlb[b], xlg.at[pl.ds(base, CH)], wlsem[b])
            dxr[b].wait()
            pltpu.async_copy(xrb[b], xrg.at[pl.ds(base, CH)], wrsem[b])
        return carry

    lax.fori_loop(0, WPE // CH // 2, pair, 0)
    for b in range(2):
        pltpu.make_async_copy(
            xlb[b], xlg.at[pl.ds(wbase, CH)], wlsem[b]).wait()
        pltpu.make_async_copy(
            xrb[b], xrg.at[pl.ds(wbase, CH)], wrsem[b]).wait()
  return _k


def _sc_gather_xlxr(xl4, xr4, src_pad, dst_pad, et_pad):
    return _build_sc_gather_xlxr()(xl4, xr4, src_pad, dst_pad, et_pad)


@functools.lru_cache(maxsize=None)
def _build_sc_scatter_den():
  @functools.partial(
    pl.kernel,
    out_type=jax.ShapeDtypeStruct((NCORE, DK, 16), jnp.float32),
    mesh=_mesh(),
    compiler_params=pltpu.CompilerParams(use_tc_tiling_on_sc=False),
    scratch_types=[pltpu.VMEM((CH, 16), jnp.float32),
                   pltpu.VMEM((CH,), jnp.int32),
                   pltpu.VMEM_SHARED((DK, 16), jnp.float32)],
  )
  def _k(pe_hbm, dkey_hbm, zeros_hbm, out_hbm, pe_v, key_v, den_sh):
    cid = lax.axis_index("c")
    sid = lax.axis_index("s")
    wid = sid * NCORE + cid
    rows = DK // NSUB
    pltpu.sync_copy(zeros_hbm.at[pl.ds(sid * rows, rows)],
                    den_sh.at[pl.ds(sid * rows, rows)])
    plsc.subcore_barrier()
    wbase = wid * (EP // NW)

    def body(i, carry):
        base = wbase + i * CH
        pltpu.sync_copy(pe_hbm.at[pl.ds(base, CH)], pe_v)
        pltpu.sync_copy(dkey_hbm.at[pl.ds(base, CH)], key_v)
        pltpu.sync_copy(pe_v, den_sh.at[key_v], add=True)
        return carry

    lax.fori_loop(0, EP // NW // CH, body, 0)
    plsc.subcore_barrier()
    pltpu.sync_copy(den_sh.at[pl.ds(sid * rows, rows)],
                    out_hbm.at[cid, pl.ds(sid * rows, rows)])
  return _k


def _sc_scatter_den(pe, dkey, den_zeros):
    return _build_sc_scatter_den()(pe, dkey, den_zeros)


@functools.lru_cache(maxsize=None)
def _build_sc_gather_den():
  @functools.partial(
    pl.kernel,
    out_type=(jax.ShapeDtypeStruct((EP, 16), jnp.float32),
              jax.ShapeDtypeStruct((EP, 16), jnp.float32)),
    mesh=_mesh(),
    compiler_params=pltpu.CompilerParams(use_tc_tiling_on_sc=False),
    scratch_types=[pltpu.VMEM((CH,), jnp.int32),
                   pltpu.VMEM((CH, 16), jnp.float32),
                   pltpu.VMEM((CH, 16), jnp.float32),
                   pltpu.SemaphoreType.DMA,
                   pltpu.SemaphoreType.DMA],
  )
  def _k(denp0, denp1, dkey_hbm, out0, out1,
         key_v, d0_v, d1_v, sem1, sem2):
    wid = lax.axis_index("s") * NCORE + lax.axis_index("c")
    wbase = wid * (EP // NW)

    def body(i, carry):
        base = wbase + i * CH
        pltpu.sync_copy(dkey_hbm.at[pl.ds(base, CH)], key_v)
        g1 = pltpu.async_copy(denp0.at[key_v], d0_v, sem1)
        g2 = pltpu.async_copy(denp1.at[key_v], d1_v, sem2)
        g1.wait()
        pltpu.sync_copy(d0_v, out0.at[pl.ds(base, CH)])
        g2.wait()
        pltpu.sync_copy(d1_v, out1.at[pl.ds(base, CH)])
        return carry

    lax.fori_loop(0, EP // NW // CH, body, 0)
  return _k


def _sc_gather_den(denp0, denp1, dkey):
    return _build_sc_gather_den()(denp0, denp1, dkey)


@functools.lru_cache(maxsize=None)
def _build_sc_scatter_hm():
  @functools.partial(
    pl.kernel,
    out_type=jax.ShapeDtypeStruct((NCORE, N, HD), jnp.float32),
    mesh=_mesh(),
    compiler_params=pltpu.CompilerParams(use_tc_tiling_on_sc=False),
    scratch_types=[pltpu.VMEM((CH, HD), jnp.float32),
                   pltpu.VMEM((CH,), jnp.int32),
                   pltpu.VMEM_SHARED((N, HD), jnp.float32)],
  )
  def _k(v_hbm, dst_hbm, zeros_hbm, out_hbm, v_v, key_v, hm_sh):
    cid = lax.axis_index("c")
    sid = lax.axis_index("s")
    wid = sid * NCORE + cid
    rows = N // NSUB
    pltpu.sync_copy(zeros_hbm.at[pl.ds(sid * rows, rows)],
                    hm_sh.at[pl.ds(sid * rows, rows)])
    plsc.subcore_barrier()
    wbase = wid * (EP // NW)

    def body(i, carry):
        base = wbase + i * CH
        pltpu.sync_copy(v_hbm.at[pl.ds(base, CH)], v_v)
        pltpu.sync_copy(dst_hbm.at[pl.ds(base, CH)], key_v)
        pltpu.sync_copy(v_v, hm_sh.at[key_v], add=True)
        return carry

    lax.fori_loop(0, EP // NW // CH, body, 0)
    plsc.subcore_barrier()
    pltpu.sync_copy(hm_sh.at[pl.ds(sid * rows, rows)],
                    out_hbm.at[cid, pl.ds(sid * rows, rows)])
  return _k


def _sc_scatter_hm(v, dst_pad, hm_zeros):
    return _build_sc_scatter_hm()(v, dst_pad, hm_zeros)


BT = 1000
BE = 1024


def _h0_body(x_ref, ide_ref, wpt_ref, bp_ref, o_ref):
    wpt = wpt_ref[...]
    z = (jnp.dot(x_ref[...], wpt[:DF], preferred_element_type=jnp.float32)
         + jnp.dot(ide_ref[...], wpt[DF:], preferred_element_type=jnp.float32)
         + bp_ref[...])
    o_ref[...] = _silu(z)


def _tc_h0(x, ide, wpt, bp2d):
    return pl.pallas_call(
        _h0_body,
        grid=(N // BT,),
        in_specs=[pl.BlockSpec((BT, DF), lambda i: (i, 0)),
                  pl.BlockSpec((BT, IDED), lambda i: (i, 0)),
                  pl.BlockSpec((DF + IDED, HD), lambda i: (0, 0)),
                  pl.BlockSpec((1, HD), lambda i: (0, 0))],
        out_specs=pl.BlockSpec((BT, HD), lambda i: (i, 0)),
        out_shape=jax.ShapeDtypeStruct((N, HD), jnp.float32),
    )(x, ide, wpt, bp2d)


def _proj_body(h_ref, wl_ref, wr_ref, xl_ref, xr_ref):
    h = h_ref[...]
    xl_ref[0] = jnp.dot(h, wl_ref[0],
                        preferred_element_type=jnp.float32).astype(jnp.bfloat16)
    xr_ref[0] = jnp.dot(h, wr_ref[0],
                        preferred_element_type=jnp.float32).astype(jnp.bfloat16)


def _tc_proj(h0, wlt, wrt):
    BTP = 2000
    return pl.pallas_call(
        _proj_body,
        grid=(NR, N // BTP),
        in_specs=[pl.BlockSpec((BTP, HD), lambda r, i: (i, 0)),
                  pl.BlockSpec((1, HD, D), lambda r, i: (r, 0, 0)),
                  pl.BlockSpec((1, HD, D), lambda r, i: (r, 0, 0))],
        out_specs=[pl.BlockSpec((1, BTP, D), lambda r, i: (r, i, 0)),
                   pl.BlockSpec((1, BTP, D), lambda r, i: (r, i, 0))],
        out_shape=(jax.ShapeDtypeStruct((NR, N, D), jnp.bfloat16),
                   jax.ShapeDtypeStruct((NR, N, D), jnp.bfloat16)),
    )(h0, wlt, wrt)


def _ef_body(ea_ref, wa_ref, rel_ref, wr_ref, et_ref, o_ref):
    dn = (((0,), (0,)), ((), ()))
    ef_all = lax.dot_general(ea_ref[...], wa_ref[...], dn,
                             preferred_element_type=jnp.float32)
    r4 = jnp.dot(rel_ref[...], wr_ref[...],
                 preferred_element_type=jnp.float32)
    et = et_ref[...]
    rr = lax.broadcasted_iota(jnp.int32, (NR, et.shape[1]), 0)
    oh = jnp.transpose((et == rr).astype(jnp.float32))
    acc = jnp.zeros((ea_ref.shape[1], D), jnp.float32)
    for r in range(NR):
        sel = ef_all[:, r * D:(r + 1) * D] + r4[r, r * D:(r + 1) * D][None]
        acc = acc + oh[:, r:r + 1] * sel
    o_ref[...] = acc.astype(jnp.bfloat16)


def _tc_ef(eat_pad, wa, rel_emb, wrel, etr):
    return pl.pallas_call(
        _ef_body,
        grid=(EP // BE,),
        in_specs=[pl.BlockSpec((DE, BE), lambda i: (0, i)),
                  pl.BlockSpec((DE, NR * D), lambda i: (0, 0)),
                  pl.BlockSpec((NR, 8), lambda i: (0, 0)),
                  pl.BlockSpec((8, NR * D), lambda i: (0, 0)),
                  pl.BlockSpec((1, BE), lambda i: (0, i))],
        out_specs=pl.BlockSpec((BE, D), lambda i: (i, 0)),
        out_shape=jax.ShapeDtypeStruct((EP, D), jnp.bfloat16),
    )(eat_pad, wa, rel_emb, wrel, etr)


def _scores_body(xlg_ref, xrg_ref, ef_ref, et_ref, att_ref, o_ref):
    pid = pl.program_id(0)
    z = (xlg_ref[...].astype(jnp.float32) + xrg_ref[...].astype(jnp.float32)
         + ef_ref[...].astype(jnp.float32))
    z = jnp.where(z >= 0, z, 0.2 * z)
    et = et_ref[...]
    rr = lax.broadcasted_iota(jnp.int32, (NR, BE), 0)
    oht = (et == rr).astype(jnp.float32)
    dn0 = (((0,), (0,)), ((), ()))
    asel = lax.dot_general(oht, att_ref[...], dn0,
                           preferred_element_type=jnp.float32)
    prod = z * asel
    f64 = lax.broadcasted_iota(jnp.int32, (D, H), 0) // HD
    hh = lax.broadcasted_iota(jnp.int32, (D, H), 1)
    hmask = (f64 == hh).astype(jnp.float32)
    dn1 = (((0,), (1,)), ((), ()))
    et_ = lax.dot_general(hmask, prod, dn1,
                          preferred_element_type=jnp.float32)
    pet = jnp.exp(et_)
    colid = pid * BE + lax.broadcasted_iota(jnp.int32, (1, BE), 1)
    pet = jnp.where(colid < E, pet, 0.0)
    o_ref[...] = jnp.concatenate(
        [pet, jnp.zeros((16 - H, BE), jnp.float32)], axis=0)


def _tc_scores(xlg, xrg, ef, etr, att4):
    return pl.pallas_call(
        _scores_body,
        grid=(EP // BE,),
        in_specs=[pl.BlockSpec((BE, D), lambda i: (i, 0)),
                  pl.BlockSpec((BE, D), lambda i: (i, 0)),
                  pl.BlockSpec((BE, D), lambda i: (i, 0)),
                  pl.BlockSpec((1, BE), lambda i: (0, i)),
                  pl.BlockSpec((NR, D), lambda i: (0, 0))],
        out_specs=pl.BlockSpec((16, BE), lambda i: (0, i)),
        out_shape=jax.ShapeDtypeStruct((16, EP), jnp.float32),
    )(xlg, xrg, ef, etr, att4)


def _values_body(xlg_ref, pet_ref, dent_ref, et_ref, rg_ref, o_ref):
    pet = pet_ref[...][:H]
    dent = dent_ref[...][:H]
    dent = jnp.where(dent > 0, dent, 1.0)
    alphat = pet / dent
    rg = rg_ref[...]
    gexp = jnp.exp(rg - jnp.max(rg, axis=1, keepdims=True))
    gw = gexp / jnp.sum(gexp, axis=1, keepdims=True)
    et = et_ref[...]
    rr = lax.broadcasted_iota(jnp.int32, (NR, BE), 0)
    oht = (et == rr).astype(jnp.float32)
    dn = (((1,), (0,)), ((), ()))
    gws = lax.dot_general(gw, oht, dn,
                          preferred_element_type=jnp.float32)
    wt = alphat * gws * (1.0 / H)
    w = jnp.transpose(wt)
    xlg = xlg_ref[...].astype(jnp.float32)
    v = jnp.zeros((BE, HD), jnp.float32)
    for h in range(H):
        v = v + w[:, h:h + 1] * xlg[:, h * HD:(h + 1) * HD]
    o_ref[...] = v


def _tc_values(xlg, pet, dent, etr, rg2d):
    return pl.pallas_call(
        _values_body,
        grid=(EP // BE,),
        in_specs=[pl.BlockSpec((BE, D), lambda i: (i, 0)),
                  pl.BlockSpec((16, BE), lambda i: (0, i)),
                  pl.BlockSpec((16, BE), lambda i: (0, i)),
                  pl.BlockSpec((1, BE), lambda i: (0, i)),
                  pl.BlockSpec((1, NR), lambda i: (0, 0))],
        out_specs=pl.BlockSpec((BE, HD), lambda i: (i, 0)),
        out_shape=jax.ShapeDtypeStruct((EP, HD), jnp.float32),
    )(xlg, pet, dent, etr, rg2d)


def _post_body(h0_ref, hmp_ref, batch_ref, rg_ref, bias4_ref,
               n1g_ref, n1b_ref, w1t_ref, b1_ref, w2t_ref, b2_ref,
               n2g_ref, n2b_ref, sums_ref, cnt_ref, maxr_ref):
    pid = pl.program_id(0)

    @pl.when(pid == 0)
    def _init():
        sums_ref[...] = jnp.zeros_like(sums_ref)
        cnt_ref[...] = jnp.zeros_like(cnt_ref)
        maxr_ref[...] = jnp.full_like(maxr_ref, -jnp.inf)

    rg = rg_ref[...]
    gexp = jnp.exp(rg - jnp.max(rg, axis=1, keepdims=True))
    gw = gexp / jnp.sum(gexp, axis=1, keepdims=True)
    hmc = jnp.dot(gw, bias4_ref[...], preferred_element_type=jnp.float32)
    h = h0_ref[...] + hmp_ref[0] + hmp_ref[1] + hmc
    h = _ln(h, n1g_ref[...], n1b_ref[...])
    hf = jnp.dot(_silu(jnp.dot(h, w1t_ref[...],
                               preferred_element_type=jnp.float32)
                       + b1_ref[...]),
                 w2t_ref[...], preferred_element_type=jnp.float32) + b2_ref[...]
    h2 = _ln(h + hf, n2g_ref[...], n2b_ref[...])

    batch = batch_ref[...]
    gg = lax.broadcasted_iota(jnp.int32, (BT, NG), 1)
    oh = (batch == gg).astype(jnp.float32)
    dn = (((0,), (0,)), ((), ()))
    sums_ref[...] += lax.dot_general(oh, h2, dn,
                                     preferred_element_type=jnp.float32)
    cnt_ref[...] += lax.dot_general(oh, jnp.ones((BT, HD), jnp.float32), dn,
                                    preferred_element_type=jnp.float32)
    for g in range(NG):
        mg = jnp.where(batch == g, h2, -jnp.inf)
        maxr_ref[g:g + 1, :] = jnp.maximum(
            maxr_ref[g:g + 1, :], jnp.max(mg, axis=0, keepdims=True))


def _tc_post(h0, hmp, batch2d, rg2d, bias4, n1g, n1b, w1t, b1, w2t, b2,
             n2g, n2b):
    return pl.pallas_call(
        _post_body,
        grid=(N // BT,),
        in_specs=[pl.BlockSpec((BT, HD), lambda i: (i, 0)),
                  pl.BlockSpec((NCORE, BT, HD), lambda i: (0, i, 0)),
                  pl.BlockSpec((BT, 1), lambda i: (i, 0)),
                  pl.BlockSpec((1, NR), lambda i: (0, 0)),
                  pl.BlockSpec((NR, HD), lambda i: (0, 0)),
                  pl.BlockSpec((1, HD), lambda i: (0, 0)),
                  pl.BlockSpec((1, HD), lambda i: (0, 0)),
                  pl.BlockSpec((HD, FH), lambda i: (0, 0)),
                  pl.BlockSpec((1, FH), lambda i: (0, 0)),
                  pl.BlockSpec((FH, HD), lambda i: (0, 0)),
                  pl.BlockSpec((1, HD), lambda i: (0, 0)),
                  pl.BlockSpec((1, HD), lambda i: (0, 0)),
                  pl.BlockSpec((1, HD), lambda i: (0, 0))],
        out_specs=[pl.BlockSpec((NG, HD), lambda i: (0, 0)),
                   pl.BlockSpec((NG, HD), lambda i: (0, 0)),
                   pl.BlockSpec((NG, HD), lambda i: (0, 0))],
        out_shape=(jax.ShapeDtypeStruct((NG, HD), jnp.float32),
                   jax.ShapeDtypeStruct((NG, HD), jnp.float32),
                   jax.ShapeDtypeStruct((NG, HD), jnp.float32)),
    )(h0, hmp, batch2d, rg2d, bias4, n1g, n1b, w1t, b1, w2t, b2, n2g, n2b)


def _kan_eval(gmat, bwt, gridt, sct):
    base = jnp.dot(_silu(gmat), bwt, preferred_element_type=jnp.float32)
    nb = gridt.shape[0] - 1
    bs = [((gmat >= gridt[j:j + 1]) & (gmat < gridt[j + 1:j + 2]))
          .astype(jnp.float32) for j in range(nb)]
    for deg in range(1, SO + 1):
        nbs = []
        for k in range(nb - deg):
            t1 = (gmat - gridt[k:k + 1]) / (gridt[k + deg:k + deg + 1]
                                            - gridt[k:k + 1])
            t2 = ((gridt[k + deg + 1:k + deg + 2] - gmat)
                  / (gridt[k + deg + 1:k + deg + 2] - gridt[k + 1:k + 2]))
            nbs.append(t1 * bs[k] + t2 * bs[k + 1])
        bs = nbs
    out = base
    for j in range(GS + SO):
        out = out + jnp.dot(bs[j], sct[j], preferred_element_type=jnp.float32)
    return out


def _kan_body(sums_ref, cnt_ref, maxr_ref, rng_ref, rnb_ref,
              bwt1_ref, gridt1_ref, sct1_ref, bwt2_ref, gridt2_ref, sct2_ref,
              o_ref):
    cnt = jnp.maximum(cnt_ref[...], 1.0)
    hmean = sums_ref[...] / cnt
    maxr = maxr_ref[...]
    hmax = jnp.where(maxr < -1e30, 0.0, maxr)
    g0 = jnp.concatenate([hmean, hmax], axis=1)
    g0 = _ln(g0, rng_ref[...], rnb_ref[...])
    g1 = _kan_eval(g0, bwt1_ref[...], gridt1_ref[...], sct1_ref[...])
    g2 = _kan_eval(g1, bwt2_ref[...], gridt2_ref[...], sct2_ref[...])
    o_ref[...] = g2


def _tc_kan(sums, cnt, maxr, rng2d, rnb2d, bwt1, gridt1, sct1,
            bwt2, gridt2, sct2):
    return pl.pallas_call(
        _kan_body,
        out_shape=jax.ShapeDtypeStruct((NG, NC), jnp.float32),
    )(sums, cnt, maxr, rng2d, rnb2d, bwt1, gridt1, sct1, bwt2, gridt2, sct2)


def kernel(x, edge_index, edge_attr, edge_type, id_token, batch, params):
    p = params
    bp0 = p['blocks'][0]
    f32 = jnp.float32

    idtok_pad = jnp.concatenate(
        [id_token.astype(jnp.int32), jnp.zeros((NP - N,), jnp.int32)])
    zpad = jnp.zeros((EP - E,), jnp.int32)
    src_pad = jnp.concatenate([edge_index[0].astype(jnp.int32), zpad])
    dst_pad = jnp.concatenate([edge_index[1].astype(jnp.int32), zpad])
    et_pad = jnp.concatenate([edge_type.astype(jnp.int32), zpad])
    etr = et_pad.reshape(1, EP)
    eat_pad = jnp.concatenate(
        [edge_attr.T, jnp.zeros((DE, EP - E), f32)], axis=1)

    wpt = p['Wp'].T
    bp2d = p['bp'].reshape(1, HD)
    wlt = jnp.stack([bp0['convs'][r]['lin_l'].T for r in range(NR)])
    wrt = jnp.stack([bp0['convs'][r]['lin_r'].T for r in range(NR)])
    wa = jnp.concatenate(
        [bp0['convs'][r]['lin_edge'][:, :DE].T for r in range(NR)], axis=1)
    wrel = jnp.concatenate(
        [bp0['convs'][r]['lin_edge'][:, DE:].T for r in range(NR)], axis=1)
    att4 = jnp.stack([bp0['convs'][r]['att'].reshape(D) for r in range(NR)])
    bias4 = jnp.stack([bp0['convs'][r]['bias'] for r in range(NR)])
    rg2d = bp0['rel_gate'].reshape(1, NR)

    den_zeros = jnp.zeros((DK, 16), f32)
    hm_zeros = jnp.zeros((N, HD), f32)

    kan1, kan2 = p['kan'][0], p['kan'][1]
    bwt1 = kan1['bw'].T
    gridt1 = kan1['grid'].T
    sct1 = jnp.transpose(kan1['sw'] * kan1['ss'][..., None], (2, 1, 0))
    bwt2 = kan2['bw'].T
    gridt2 = kan2['grid'].T
    sct2 = jnp.transpose(kan2['sw'] * kan2['ss'][..., None], (2, 1, 0))

    ide = _sc_gather_ide(p['id_emb'], idtok_pad)[:N]
    h0 = _tc_h0(x, ide, wpt, bp2d)
    xl, xr = _tc_proj(h0, wlt, wrt)
    xl4 = xl.reshape(NR * N, D)
    xr4 = xr.reshape(NR * N, D)
    ef = _tc_ef(eat_pad, wa, bp0['rel_emb'], wrel, etr)
    xlg, xrg, dkey = _sc_gather_xlxr(xl4, xr4, src_pad, dst_pad, et_pad)
    pet = _tc_scores(xlg, xrg, ef, etr, att4)
    pe_lin = jnp.transpose(pet)
    denp = _sc_scatter_den(pe_lin, dkey, den_zeros)
    dg0, dg1 = _sc_gather_den(denp[0], denp[1], dkey)
    dent = jnp.transpose(dg0 + dg1)
    v = _tc_values(xlg, pet, dent, etr, rg2d)
    hmp = _sc_scatter_hm(v, dst_pad, hm_zeros)
    sums, cnt, maxr = _tc_post(
        h0, hmp, batch.astype(jnp.int32).reshape(N, 1), rg2d, bias4,
        bp0['n1g'].reshape(1, HD), bp0['n1b'].reshape(1, HD),
        bp0['W1'].T, bp0['b1'].reshape(1, FH),
        bp0['W2'].T, bp0['b2'].reshape(1, HD),
        bp0['n2g'].reshape(1, HD), bp0['n2b'].reshape(1, HD))
    out = _tc_kan(sums, cnt, maxr,
                  p['rng'].reshape(1, 2 * HD), p['rnb'].reshape(1, 2 * HD),
                  bwt1, gridt1, sct1, bwt2, gridt2, sct2)
    return out
